# baseline (speedup 1.0000x reference)
import jax
import jax.numpy as jnp
from jax.experimental import pallas as pl
from jax.experimental.pallas import tpu as pltpu

_C = 64
_OUT = 64
_BATCHES_PER_PROGRAM = 8


def _body(x_ref, wr_ref, wi_ref, o_ref):
    w_eff = 2.0 * (wr_ref[0:64, :] - wi_ref[64:128, :] - wr_ref[128:192, :]
                   + wi_ref[192:256, :] + wr_ref[256:320, :])
    xb = x_ref[0]
    for b in range(o_ref.shape[0]):
        o_ref[b] = jax.lax.dot_general(
            xb[b * _C:(b + 1) * _C], w_eff, (((0,), (0,)), ((), ())),
            preferred_element_type=jnp.float32)


def kernel(x, real_weights, imag_weights):
    N, C, m, _ = x.shape
    M = m * m
    B = _BATCHES_PER_PROGRAM if N % _BATCHES_PER_PROGRAM == 0 else 1
    xr = x.reshape(N // B, B * C, M)
    out = pl.pallas_call(
        _body,
        grid=(N // B,),
        in_specs=[
            pl.BlockSpec((1, B * C, M), lambda n: (n, 0, 0)),
            pl.BlockSpec(real_weights.shape, lambda n: (0, 0)),
            pl.BlockSpec(imag_weights.shape, lambda n: (0, 0)),
        ],
        out_specs=pl.BlockSpec((B, M, _OUT), lambda n: (n, 0, 0)),
        out_shape=jax.ShapeDtypeStruct((N, M, _OUT), jnp.float32),
        compiler_params=pltpu.CompilerParams(
            dimension_semantics=("parallel",)),
    )(xr, real_weights, imag_weights)
    return out.reshape(N, m, m, _OUT)


# speedup vs baseline: 1.0659x; 1.0659x over previous
import jax
import jax.numpy as jnp
from jax.experimental import pallas as pl
from jax.experimental.pallas import tpu as pltpu

_C = 64
_OUT = 64
_BATCHES_PER_PROGRAM = 4
_M_SPLIT = 2


def _body(x_ref, wr_ref, wi_ref, o_ref):
    w_eff = 2.0 * (wr_ref[0:64, :] - wi_ref[64:128, :] - wr_ref[128:192, :]
                   + wi_ref[192:256, :] + wr_ref[256:320, :])
    xb = x_ref[0]
    for b in range(o_ref.shape[0]):
        o_ref[b] = jax.lax.dot_general(
            xb[b * _C:(b + 1) * _C], w_eff, (((0,), (0,)), ((), ())),
            preferred_element_type=jnp.float32)


def kernel(x, real_weights, imag_weights):
    N, C, m, _ = x.shape
    M = m * m
    B = _BATCHES_PER_PROGRAM if N % _BATCHES_PER_PROGRAM == 0 else 1
    S = _M_SPLIT if M % _M_SPLIT == 0 else 1
    xr = x.reshape(N // B, B * C, M)
    out = pl.pallas_call(
        _body,
        grid=(N // B, S),
        in_specs=[
            pl.BlockSpec((1, B * C, M // S), lambda n, j: (n, 0, j)),
            pl.BlockSpec(real_weights.shape, lambda n, j: (0, 0)),
            pl.BlockSpec(imag_weights.shape, lambda n, j: (0, 0)),
        ],
        out_specs=pl.BlockSpec((B, M // S, _OUT), lambda n, j: (n, j, 0)),
        out_shape=jax.ShapeDtypeStruct((N, M, _OUT), jnp.float32),
        compiler_params=pltpu.CompilerParams(
            dimension_semantics=("parallel", "parallel")),
    )(xr, real_weights, imag_weights)
    return out.reshape(N, m, m, _OUT)


# speedup vs baseline: 1.5720x; 1.4748x over previous
import jax
import jax.numpy as jnp
from jax.experimental import pallas as pl
from jax.experimental.pallas import tpu as pltpu

_C = 64
_OUT = 64


def _body(x_ref, wr_ref, wi_ref, o_ref):
    w_eff = 2.0 * (wr_ref[0:64, :] - wi_ref[64:128, :] - wr_ref[128:192, :]
                   + wi_ref[192:256, :] + wr_ref[256:320, :])
    xb = x_ref[0]
    for i in range(xb.shape[1]):
        o_ref[0, i] = jax.lax.dot_general(
            xb[:, i, :], w_eff, (((0,), (0,)), ((), ())),
            preferred_element_type=jnp.float32)


def kernel(x, real_weights, imag_weights):
    N, C, m, _ = x.shape
    out = pl.pallas_call(
        _body,
        grid=(N,),
        in_specs=[
            pl.BlockSpec((1, C, m, m), lambda n: (n, 0, 0, 0)),
            pl.BlockSpec(real_weights.shape, lambda n: (0, 0)),
            pl.BlockSpec(imag_weights.shape, lambda n: (0, 0)),
        ],
        out_specs=pl.BlockSpec((1, m, m, _OUT), lambda n: (n, 0, 0, 0)),
        out_shape=jax.ShapeDtypeStruct((N, m, m, _OUT), jnp.float32),
        compiler_params=pltpu.CompilerParams(
            dimension_semantics=("parallel",)),
    )(x, real_weights, imag_weights)
    return out


# speedup vs baseline: 1.6945x; 1.0779x over previous
import jax
import jax.numpy as jnp
from jax.experimental import pallas as pl
from jax.experimental.pallas import tpu as pltpu

_C = 64
_OUT = 64


_B = 2


def _body(x_ref, wr_ref, wi_ref, o_ref):
    w_eff = 2.0 * (wr_ref[0:64, :] - wi_ref[64:128, :] - wr_ref[128:192, :]
                   + wi_ref[192:256, :] + wr_ref[256:320, :])
    for b in range(o_ref.shape[0]):
        xb = x_ref[b]
        for i in range(xb.shape[1]):
            o_ref[b, i] = jax.lax.dot_general(
                xb[:, i, :], w_eff, (((0,), (0,)), ((), ())),
                preferred_element_type=jnp.float32)


def kernel(x, real_weights, imag_weights):
    N, C, m, _ = x.shape
    B = _B if N % _B == 0 else 1
    out = pl.pallas_call(
        _body,
        grid=(N // B,),
        in_specs=[
            pl.BlockSpec((B, C, m, m), lambda n: (n, 0, 0, 0)),
            pl.BlockSpec(real_weights.shape, lambda n: (0, 0)),
            pl.BlockSpec(imag_weights.shape, lambda n: (0, 0)),
        ],
        out_specs=pl.BlockSpec((B, m, m, _OUT), lambda n: (n, 0, 0, 0)),
        out_shape=jax.ShapeDtypeStruct((N, m, m, _OUT), jnp.float32),
        compiler_params=pltpu.CompilerParams(
            dimension_semantics=("parallel",)),
    )(x, real_weights, imag_weights)
    return out
